# baseline (speedup 1.0000x reference)
import functools

import jax
import jax.numpy as jnp
from jax import lax
from jax.experimental import pallas as pl
from jax.experimental.pallas import tpu as pltpu
from jax.experimental.pallas import tpu_sc as plsc

_D = 64
_B = 16384
_V = 100000
_NC = 2
_NS = 16
_NW = _NC * _NS
_BPW = _B // _NW
_PPW = _BPW // 2
_CH = 128
_NCH = _BPW // _CH
_VFULL = _V // 128
_VTAIL = _V - _VFULL * 128


def _transpose_chunk(cb, sg):
    iotas = [lax.iota(jnp.int32, 16) + k * 16 for k in range(_D // 16)]
    for j in range(64):
        for k in range(_D // 16):
            lo = plsc.load_gather(cb, [iotas[k], jnp.full((16,), 2 * j,
                                                          jnp.int32)])
            hi = plsc.load_gather(cb, [iotas[k], jnp.full((16,), 2 * j + 1,
                                                          jnp.int32)])
            sg[j, pl.ds(k * 16, 16)] = lo
            sg[j, pl.ds(_D + k * 16, 16)] = hi


def _k1_body(embt_hbm, tail_hbm, out_hbm, cb, sg, tl, sem):
    c = lax.axis_index("c")
    s = lax.axis_index("s")
    wid = s * _NC + c

    def do_chunk(t):
        ch = wid + t * _NW
        pltpu.sync_copy(embt_hbm.at[:, pl.ds(ch * 128, 128)], cb)
        _transpose_chunk(cb, sg)
        pltpu.sync_copy(sg, out_hbm.at[pl.ds(ch * 64, 64)])

    def body(t, carry):
        ch = wid + t * _NW

        @pl.when(ch < _VFULL)
        def _():
            do_chunk(t)
        return carry

    nt = (_VFULL + _NW - 1) // _NW
    lax.fori_loop(0, nt, body, 0)

    @pl.when(wid == _NW - 1)
    def _():
        pltpu.sync_copy(tail_hbm, tl)
        pltpu.sync_copy(tl, out_hbm.at[pl.ds(_VFULL * 64, 16)])


@functools.partial(
    pl.kernel,
    out_type=jax.ShapeDtypeStruct((_V // 2, 2 * _D), jnp.float32),
    mesh=plsc.VectorSubcoreMesh(core_axis_name="c", subcore_axis_name="s"),
    compiler_params=pltpu.CompilerParams(use_tc_tiling_on_sc=True,
                                         needs_layout_passes=False),
    scratch_types=[
        pltpu.VMEM((_D, 128), jnp.float32),
        pltpu.VMEM((64, 2 * _D), jnp.float32),
        pltpu.VMEM((16, 2 * _D), jnp.float32),
        pltpu.SemaphoreType.DMA,
    ],
)
def _k1(embt_hbm, tail_hbm, out_hbm, cb, sg, tl, sem):
    _k1_body(embt_hbm, tail_hbm, out_hbm, cb, sg, tl, sem)


def _k2_body(rep_hbm, idx2_hbm, par_hbm, table_hbm, out_hbm,
             idx_v, par_v, rows_v, rep_v, acc_v, sem_g, sem_r):
    c = lax.axis_index("c")
    s = lax.axis_index("s")
    wid = s * _NC + c
    base = wid * _BPW

    pltpu.sync_copy(idx2_hbm.at[pl.ds(base, _BPW)], idx_v)
    pltpu.sync_copy(par_hbm.at[pl.ds(base, _BPW)], par_v)
    rep_cp = pltpu.async_copy(rep_hbm.at[pl.ds(wid * _PPW, _PPW)], rep_v,
                              sem_r)
    gathers = []
    for j in range(_NCH):
        gathers.append(pltpu.async_copy(
            table_hbm.at[idx_v.at[pl.ds(j * _CH, _CH)]],
            rows_v.at[pl.ds(j * _CH, _CH)], sem_g))
    rep_cp.wait()
    for g in gathers:
        g.wait()

    nk = _D // 16

    def one_row(b, rep_row, rep_off, accs):
        pbc = plsc.load_gather(par_v, [jnp.full((16,), b, jnp.int32)])
        hi = pbc != 0
        new = []
        for k in range(nk):
            e_lo = rows_v[b, pl.ds(k * 16, 16)]
            e_hi = rows_v[b, pl.ds(_D + k * 16, 16)]
            e = jnp.where(hi, e_hi, e_lo)
            r = rep_v[rep_row, pl.ds(rep_off + k * 16, 16)]
            d = e - r
            new.append(accs[k] + d * d)
        return tuple(new)

    def body(j, accs):
        accs = one_row(2 * j, j, 0, accs)
        accs = one_row(2 * j + 1, j, _D, accs)
        return accs

    zero = jnp.zeros((16,), jnp.float32)
    accs = lax.fori_loop(0, _PPW, body, (zero,) * nk)
    total = accs[0]
    for a in accs[1:]:
        total = total + a
    acc_v[...] = total * (1.0 / (_B * _D))
    pltpu.sync_copy(acc_v, out_hbm.at[wid])


@functools.partial(
    pl.kernel,
    out_type=jax.ShapeDtypeStruct((_NW, 16), jnp.float32),
    mesh=plsc.VectorSubcoreMesh(core_axis_name="c", subcore_axis_name="s"),
    compiler_params=pltpu.CompilerParams(use_tc_tiling_on_sc=True,
                                         needs_layout_passes=False),
    scratch_types=[
        pltpu.VMEM((_BPW,), jnp.int32),
        pltpu.VMEM((_BPW,), jnp.int32),
        pltpu.VMEM((_BPW, 2 * _D), jnp.float32),
        pltpu.VMEM((_PPW, 2 * _D), jnp.float32),
        pltpu.VMEM((16,), jnp.float32),
        pltpu.SemaphoreType.DMA,
        pltpu.SemaphoreType.DMA,
    ],
)
def _k2(rep_hbm, idx2_hbm, par_hbm, table_hbm, out_hbm,
        idx_v, par_v, rows_v, rep_v, acc_v, sem_g, sem_r):
    _k2_body(rep_hbm, idx2_hbm, par_hbm, table_hbm, out_hbm,
             idx_v, par_v, rows_v, rep_v, acc_v, sem_g, sem_r)


def kernel(rep, expr, emb_weight):
    idx = expr.astype(jnp.int32)
    embt = emb_weight.T
    tail_pairs = emb_weight[_VFULL * 128:, :].reshape(_VTAIL // 2, 2 * _D)
    table2 = _k1(embt, tail_pairs)
    partials = _k2(rep.reshape(_B // 2, 2 * _D), idx >> 1, idx & 1, table2)
    return jnp.sum(partials)


# speedup vs baseline: 1.2296x; 1.2296x over previous
import functools

import jax
import jax.numpy as jnp
from jax import lax
from jax.experimental import pallas as pl
from jax.experimental.pallas import tpu as pltpu
from jax.experimental.pallas import tpu_sc as plsc

_D = 64
_B = 16384
_V = 100000
_NC = 2
_NS = 16
_NW = _NC * _NS
_BPW = _B // _NW
_PPW = _BPW // 2
_CH = 128
_NCH = _BPW // _CH
_VFULL = _V // 128
_VTAIL = _V - _VFULL * 128


_PITCH = 129


def _transpose_chunk(cb, sg):
    iotas = [lax.iota(jnp.int32, 16) + k * 16 for k in range(_D // 16)]

    def body(j, carry):
        v0 = jnp.full((16,), 2 * j, jnp.int32)
        v1 = v0 + 1
        for k in range(_D // 16):
            lo = plsc.load_gather(cb, [iotas[k], v0])
            hi = plsc.load_gather(cb, [iotas[k], v1])
            sg[j, pl.ds(k * 16, 16)] = lo
            sg[j, pl.ds(_D + k * 16, 16)] = hi
        return carry

    lax.fori_loop(0, 64, body, 0)


def _k1_body(embt_hbm, tail_hbm, out_hbm, cb0, cb1, sg0, sg1, tl,
             si0, si1, so0, so1):
    c = lax.axis_index("c")
    s = lax.axis_index("s")
    wid = s * _NC + c

    def start_in(t, cb, sem):
        ch = wid + t * _NW
        pltpu.async_copy(embt_hbm.at[:, pl.ds(ch * 128, 128)],
                         cb.at[:, pl.ds(0, 128)], sem)

    def wait_in(cb, sem):
        pltpu.make_async_copy(embt_hbm.at[:, pl.ds(0, 128)],
                              cb.at[:, pl.ds(0, 128)], sem).wait()

    def start_out(t, sg, sem):
        ch = wid + t * _NW
        pltpu.async_copy(sg, out_hbm.at[pl.ds(ch * 64, 64)], sem)

    def wait_out(sg, sem):
        pltpu.make_async_copy(sg, out_hbm.at[pl.ds(0, 64)], sem).wait()

    nt = (_VFULL + _NW - 1) // _NW
    ng = (nt - 1) // 2

    start_in(0, cb0, si0)

    def body(g, carry):
        start_in(2 * g + 1, cb1, si1)
        wait_in(cb0, si0)

        @pl.when(g >= 1)
        def _():
            wait_out(sg0, so0)
        _transpose_chunk(cb0, sg0)
        start_out(2 * g, sg0, so0)

        @pl.when(wid + (2 * g + 2) * _NW < _VFULL)
        def _():
            start_in(2 * g + 2, cb0, si0)
        wait_in(cb1, si1)

        @pl.when(g >= 1)
        def _():
            wait_out(sg1, so1)
        _transpose_chunk(cb1, sg1)
        start_out(2 * g + 1, sg1, so1)
        return carry

    lax.fori_loop(0, ng, body, 0)

    @pl.when(wid + (nt - 1) * _NW < _VFULL)
    def _():
        wait_in(cb0, si0)
        wait_out(sg0, so0)
        _transpose_chunk(cb0, sg0)
        start_out(nt - 1, sg0, so0)

    wait_out(sg0, so0)
    wait_out(sg1, so1)

    @pl.when(wid == _NW - 1)
    def _():
        pltpu.sync_copy(tail_hbm, tl)
        pltpu.sync_copy(tl, out_hbm.at[pl.ds(_VFULL * 64, 16)])


@functools.partial(
    pl.kernel,
    out_type=jax.ShapeDtypeStruct((_V // 2, 2 * _D), jnp.float32),
    mesh=plsc.VectorSubcoreMesh(core_axis_name="c", subcore_axis_name="s"),
    compiler_params=pltpu.CompilerParams(use_tc_tiling_on_sc=True,
                                         needs_layout_passes=False),
    scratch_types=[
        pltpu.VMEM((_D, _PITCH), jnp.float32),
        pltpu.VMEM((_D, _PITCH), jnp.float32),
        pltpu.VMEM((64, 2 * _D), jnp.float32),
        pltpu.VMEM((64, 2 * _D), jnp.float32),
        pltpu.VMEM((16, 2 * _D), jnp.float32),
        pltpu.SemaphoreType.DMA,
        pltpu.SemaphoreType.DMA,
        pltpu.SemaphoreType.DMA,
        pltpu.SemaphoreType.DMA,
    ],
)
def _k1(embt_hbm, tail_hbm, out_hbm, cb0, cb1, sg0, sg1, tl,
        si0, si1, so0, so1):
    _k1_body(embt_hbm, tail_hbm, out_hbm, cb0, cb1, sg0, sg1, tl,
             si0, si1, so0, so1)


def _k2_body(rep_hbm, idx2_hbm, par_hbm, table_hbm, out_hbm,
             idx_v, par_v, rows_v, rep_v, acc_v, sem_g, sem_r):
    c = lax.axis_index("c")
    s = lax.axis_index("s")
    wid = s * _NC + c
    base = wid * _BPW

    pltpu.sync_copy(idx2_hbm.at[pl.ds(base, _BPW)], idx_v)
    pltpu.sync_copy(par_hbm.at[pl.ds(base, _BPW)], par_v)
    rep_cp = pltpu.async_copy(rep_hbm.at[pl.ds(wid * _PPW, _PPW)], rep_v,
                              sem_r)
    gathers = []
    for j in range(_NCH):
        gathers.append(pltpu.async_copy(
            table_hbm.at[idx_v.at[pl.ds(j * _CH, _CH)]],
            rows_v.at[pl.ds(j * _CH, _CH)], sem_g))
    rep_cp.wait()
    for g in gathers:
        g.wait()

    nk = _D // 16

    def one_row(b, rep_row, rep_off, accs):
        pbc = plsc.load_gather(par_v, [jnp.full((16,), b, jnp.int32)])
        hi = pbc != 0
        new = []
        for k in range(nk):
            e_lo = rows_v[b, pl.ds(k * 16, 16)]
            e_hi = rows_v[b, pl.ds(_D + k * 16, 16)]
            e = jnp.where(hi, e_hi, e_lo)
            r = rep_v[rep_row, pl.ds(rep_off + k * 16, 16)]
            d = e - r
            new.append(accs[k] + d * d)
        return tuple(new)

    def body(j, accs):
        accs = one_row(2 * j, j, 0, accs)
        accs = one_row(2 * j + 1, j, _D, accs)
        return accs

    zero = jnp.zeros((16,), jnp.float32)
    accs = lax.fori_loop(0, _PPW, body, (zero,) * nk)
    total = accs[0]
    for a in accs[1:]:
        total = total + a
    acc_v[...] = total * (1.0 / (_B * _D))
    pltpu.sync_copy(acc_v, out_hbm.at[wid])


@functools.partial(
    pl.kernel,
    out_type=jax.ShapeDtypeStruct((_NW, 16), jnp.float32),
    mesh=plsc.VectorSubcoreMesh(core_axis_name="c", subcore_axis_name="s"),
    compiler_params=pltpu.CompilerParams(use_tc_tiling_on_sc=True,
                                         needs_layout_passes=False),
    scratch_types=[
        pltpu.VMEM((_BPW,), jnp.int32),
        pltpu.VMEM((_BPW,), jnp.int32),
        pltpu.VMEM((_BPW, 2 * _D), jnp.float32),
        pltpu.VMEM((_PPW, 2 * _D), jnp.float32),
        pltpu.VMEM((16,), jnp.float32),
        pltpu.SemaphoreType.DMA,
        pltpu.SemaphoreType.DMA,
    ],
)
def _k2(rep_hbm, idx2_hbm, par_hbm, table_hbm, out_hbm,
        idx_v, par_v, rows_v, rep_v, acc_v, sem_g, sem_r):
    _k2_body(rep_hbm, idx2_hbm, par_hbm, table_hbm, out_hbm,
             idx_v, par_v, rows_v, rep_v, acc_v, sem_g, sem_r)


def kernel(rep, expr, emb_weight):
    idx = expr.astype(jnp.int32)
    embt = emb_weight.T
    tail_pairs = emb_weight[_VFULL * 128:, :].reshape(_VTAIL // 2, 2 * _D)
    table2 = _k1(embt, tail_pairs)
    partials = _k2(rep.reshape(_B // 2, 2 * _D), idx >> 1, idx & 1, table2)
    return jnp.sum(partials)
